# baseline (speedup 1.0000x reference)
import functools

import jax
import jax.numpy as jnp
from jax import lax
from jax.experimental import pallas as pl
from jax.experimental.pallas import tpu as pltpu
from jax.experimental.pallas import tpu_sc as plsc

S = 512
L = 16
T = 256
C_CODES = 8
BB = 4
NC = 2
NS = 16
NW = NC * NS


def _col2row(x_col, diag):
    zero = jnp.zeros((), x_col.dtype)
    return jnp.sum(jnp.where(diag, x_col, zero), axis=0, keepdims=True)


def _row2col(x_row, diag):
    zero = jnp.zeros((), x_row.dtype)
    return jnp.sum(jnp.where(diag, x_row, zero), axis=1, keepdims=True)


def _sum_t_like_xla(x):
    a = x[:, 0:128] + x[:, 128:256]
    at = jnp.transpose(a)
    acc = at[0:8, :]
    for k in range(1, 16):
        acc = acc + at[8 * k:8 * (k + 1), :]
    b1 = acc[0:4, :] + acc[4:8, :]
    b2 = b1[0:2, :] + b1[2:4, :]
    return b2[0:1, :] + b2[1:2, :]


def _rank_kernel(num_codes_ref, att_ref, sent_ref, sent_t_ref, len_col_ref,
                 att_out_ref, rank_out_ref, cid_out_ref, vc_out_ref):
    pid = pl.program_id(0)

    iota_sub = lax.broadcasted_iota(jnp.int32, (S, S), 0)
    iota_lane = lax.broadcasted_iota(jnp.int32, (S, S), 1)
    diag = iota_sub == iota_lane
    tri = iota_sub < iota_lane

    for bb in range(BB):
        a = sent_ref[bb]
        at = sent_t_ref[bb]
        acc = None
        for w in range(L // 2):
            p_col = a[:, 2 * w:2 * w + 1] * 32768 + a[:, 2 * w + 1:2 * w + 2]
            p_row = (at[2 * w:2 * w + 1, :] * 32768
                     + at[2 * w + 1:2 * w + 2, :])
            eq_w = p_col == p_row
            acc = eq_w if acc is None else (acc & eq_w)
        cid_out_ref[bb] = jnp.min(jnp.where(acc, iota_sub, S), axis=0,
                                  keepdims=True)

        len_col = len_col_ref[bb]
        mask0_row = _col2row(len_col, diag) == 0
        num_sent = jnp.sum(jnp.where(len_col != 0, 1, 0))
        ncodes = num_codes_ref[pid * BB + bb]

        for c in range(C_CODES):
            att_sum_row = _sum_t_like_xla(att_ref[bb, c])
            att_row = jnp.where(mask0_row, -1.0, att_sum_row)
            att_col = _row2col(att_row, diag)

            gt = att_col > att_row
            ge = att_col >= att_row
            before = gt | (tri & ge)
            rank_row = jnp.sum(before.astype(jnp.int32), axis=0,
                               keepdims=True)

            att_out_ref[bb, c] = att_row
            rank_out_ref[bb, c] = rank_row
            vc = jnp.where(c < ncodes, num_sent, 0)
            vc_out_ref[bb, c] = jnp.full((1, 1), vc, jnp.int32)


def _sc_group_kernel(rank_hbm, cid_hbm, vc_hbm, sorted_hbm, group_hbm,
                     rank_v, cid_v, sorted_v, ordered_v, group_v, gtab_v,
                     tmp_v, vc_v):
    core = lax.axis_index("c")
    sub = lax.axis_index("s")
    wid = sub * NC + core
    pltpu.sync_copy(vc_hbm, vc_v)
    iota16 = lax.iota(jnp.int32, 16)
    minus1 = jnp.full((16,), -1, jnp.int32)

    for k in range(2):
        t = wid * 2 + k
        b = t // 8
        pltpu.sync_copy(rank_hbm.at[t], rank_v)
        pltpu.sync_copy(cid_hbm.at[b], cid_v)

        def init_body(j, carry):
            gtab_v[pl.ds(j * 16, 16)] = minus1
            return carry

        lax.fori_loop(0, S // 16, init_body, 0)

        def scat_body(j, carry):
            idx = rank_v[pl.ds(j * 16, 16)]
            plsc.store_scatter(sorted_v, [idx], iota16 + j * 16)
            plsc.store_scatter(ordered_v, [idx], cid_v[pl.ds(j * 16, 16)])
            return carry

        lax.fori_loop(0, S // 16, scat_body, 0)

        vc_t = plsc.load_gather(vc_v, [jnp.full((16,), t, jnp.int32)])

        def grp_body(j, ng):
            ov = ordered_v[pl.ds(j * 16, 16)]
            key = ov * 16 + iota16
            ks = lax.sort(key)
            sv = jnp.right_shift(ks, 4)
            sl = ks & 15
            tmp_v[...] = sv
            prev = plsc.load_gather(tmp_v, [(iota16 + 15) & 15])
            dup_sorted = jnp.where((sv == prev) & (iota16 >= 1), 1, 0)
            plsc.store_scatter(tmp_v, [sl], dup_sorted)
            dup = tmp_v[...] != 0
            g0 = plsc.load_gather(gtab_v, [ov])
            avail = (g0 == -1) & (~dup)
            avail_i = avail.astype(jnp.int32)
            excl = plsc.cumsum(avail_i) - avail_i
            gid_new = ng + excl
            plsc.store_scatter(gtab_v, [ov], gid_new, mask=avail)
            g1 = plsc.load_gather(gtab_v, [ov])
            r_vec = iota16 + j * 16
            group_v[pl.ds(j * 16, 16)] = jnp.where(r_vec < vc_t, g1, -1)
            return ng + jnp.max(plsc.all_reduce_population_count(avail))

        lax.fori_loop(0, S // 16, grp_body, jnp.int32(0))

        pltpu.sync_copy(sorted_v, sorted_hbm.at[t])
        pltpu.sync_copy(group_v, group_hbm.at[t])


@jax.jit
def kernel(article_sentences, article_sentences_lengths, attention, num_codes):
    B, S_, L_ = article_sentences.shape
    C = attention.shape[1]
    sent = article_sentences.astype(jnp.int32)
    sent_t = jnp.swapaxes(sent, 1, 2)
    lengths = article_sentences_lengths.astype(jnp.int32)
    len_col = lengths.reshape(B, S_, 1)

    grid = (B // BB,)
    row_spec = pl.BlockSpec((BB, C, 1, S_), lambda b: (b, 0, 0, 0))
    att_s, rank_s, cid_s, vc_s = pl.pallas_call(
        _rank_kernel,
        grid=grid,
        in_specs=[
            pl.BlockSpec(memory_space=pltpu.SMEM),
            pl.BlockSpec((BB, C, S_, T), lambda b: (b, 0, 0, 0)),
            pl.BlockSpec((BB, S_, L_), lambda b: (b, 0, 0)),
            pl.BlockSpec((BB, L_, S_), lambda b: (b, 0, 0)),
            pl.BlockSpec((BB, S_, 1), lambda b: (b, 0, 0)),
        ],
        out_specs=(row_spec, row_spec,
                   pl.BlockSpec((BB, 1, S_), lambda b: (b, 0, 0)),
                   pl.BlockSpec((BB, C, 1, 1), lambda b: (b, 0, 0, 0))),
        out_shape=(
            jax.ShapeDtypeStruct((B, C, 1, S_), jnp.float32),
            jax.ShapeDtypeStruct((B, C, 1, S_), jnp.int32),
            jax.ShapeDtypeStruct((B, 1, S_), jnp.int32),
            jax.ShapeDtypeStruct((B, C, 1, 1), jnp.int32),
        ),
    )(num_codes.astype(jnp.int32), attention, sent, sent_t, len_col)

    rank2 = rank_s.reshape(B * C, S_)
    cid2 = cid_s.reshape(B, S_)
    vc2 = vc_s.reshape(B * C)

    sc_fn = pl.kernel(
        _sc_group_kernel,
        out_type=(
            jax.ShapeDtypeStruct((B * C, S_), jnp.int32),
            jax.ShapeDtypeStruct((B * C, S_), jnp.int32),
        ),
        mesh=plsc.VectorSubcoreMesh(core_axis_name="c", subcore_axis_name="s",
                                    num_cores=NC, num_subcores=NS),
        scratch_types=[
            pltpu.VMEM((S_,), jnp.int32),
            pltpu.VMEM((S_,), jnp.int32),
            pltpu.VMEM((S_,), jnp.int32),
            pltpu.VMEM((S_,), jnp.int32),
            pltpu.VMEM((S_,), jnp.int32),
            pltpu.VMEM((S_,), jnp.int32),
            pltpu.VMEM((16,), jnp.int32),
            pltpu.VMEM((B * C,), jnp.int32),
        ],
        compiler_params=pltpu.CompilerParams(needs_layout_passes=False),
    )
    sorted2, group2 = sc_fn(rank2, cid2, vc2)

    return (att_s.reshape(B, C, S_), sorted2.reshape(B, C, S_),
            group2.reshape(B, C, S_))


# speedup vs baseline: 1.0465x; 1.0465x over previous
import jax
import jax.numpy as jnp
from jax import lax
from jax.experimental import pallas as pl
from jax.experimental.pallas import tpu as pltpu
from jax.experimental.pallas import tpu_sc as plsc

S = 512
L = 16
T = 256
C_CODES = 8
BB = 2
NC = 2
NS = 16


def _col2row(x_col, diag):
    zero = jnp.zeros((), x_col.dtype)
    return jnp.sum(jnp.where(diag, x_col, zero), axis=0, keepdims=True)


def _row2col(x_row, diag):
    zero = jnp.zeros((), x_row.dtype)
    return jnp.sum(jnp.where(diag, x_row, zero), axis=1, keepdims=True)


def _sum_t_like_xla(x):
    a = x[:, 0:128] + x[:, 128:256]
    at = jnp.transpose(a)
    acc = at[0:8, :]
    for k in range(1, 16):
        acc = acc + at[8 * k:8 * (k + 1), :]
    b1 = acc[0:4, :] + acc[4:8, :]
    b2 = b1[0:2, :] + b1[2:4, :]
    return b2[0:1, :] + b2[1:2, :]


def _rank_kernel(num_codes_ref, att_ref, sent_ref, sent_t_ref, len_col_ref,
                 att_out_ref, rank_out_ref, cid_out_ref, vc_out_ref):
    pid = pl.program_id(0)

    iota_sub = lax.broadcasted_iota(jnp.int32, (S, S), 0)
    iota_lane = lax.broadcasted_iota(jnp.int32, (S, S), 1)
    diag = iota_sub == iota_lane
    tri = iota_sub < iota_lane

    for bb in range(BB):
        a = sent_ref[bb]
        at = sent_t_ref[bb]
        acc = None
        for w in range(L // 2):
            p_col = a[:, 2 * w:2 * w + 1] * 32768 + a[:, 2 * w + 1:2 * w + 2]
            p_row = (at[2 * w:2 * w + 1, :] * 32768
                     + at[2 * w + 1:2 * w + 2, :])
            eq_w = p_col == p_row
            acc = eq_w if acc is None else (acc & eq_w)
        cid_out_ref[bb] = jnp.min(jnp.where(acc, iota_sub, S), axis=0,
                                  keepdims=True)

        len_col = len_col_ref[bb]
        mask0_row = _col2row(len_col, diag) == 0
        num_sent = jnp.sum(jnp.where(len_col != 0, 1, 0))
        ncodes = num_codes_ref[pid * BB + bb]

        for c in range(C_CODES):
            att_sum_row = _sum_t_like_xla(att_ref[bb, c])
            att_row = jnp.where(mask0_row, -1.0, att_sum_row)
            att_col = _row2col(att_row, diag)

            gt = att_col > att_row
            ge = att_col >= att_row
            before = gt | (tri & ge)
            rank_row = jnp.sum(before.astype(jnp.int32), axis=0,
                               keepdims=True)

            att_out_ref[bb, c] = att_row
            rank_out_ref[bb, c] = rank_row
            vc = jnp.where(c < ncodes, num_sent, 0)
            vc_out_ref[bb, c] = jnp.full((1, 1), vc, jnp.int32)


def _sc_group_kernel(rank_hbm, cid_hbm, vc_hbm, sorted_hbm, group_hbm,
                     rank_v, cid_v, sorted_v, ordered_v, group_v, gtab_v,
                     tmp_v, vc_v):
    core = lax.axis_index("c")
    sub = lax.axis_index("s")
    wid = sub * NC + core
    pltpu.sync_copy(vc_hbm, vc_v)
    iota16 = lax.iota(jnp.int32, 16)
    minus1 = jnp.full((16,), -1, jnp.int32)

    for k in range(2):
        t = wid * 2 + k
        b = t // 8
        pltpu.sync_copy(rank_hbm.at[t], rank_v)
        pltpu.sync_copy(cid_hbm.at[b], cid_v)

        def init_body(j, carry):
            gtab_v[pl.ds(j * 16, 16)] = minus1
            return carry

        lax.fori_loop(0, S // 16, init_body, 0)

        def scat_body(j, carry):
            idx = rank_v[pl.ds(j * 16, 16)]
            plsc.store_scatter(sorted_v, [idx], iota16 + j * 16)
            plsc.store_scatter(ordered_v, [idx], cid_v[pl.ds(j * 16, 16)])
            return carry

        lax.fori_loop(0, S // 16, scat_body, 0)

        vc_t = plsc.load_gather(vc_v, [jnp.full((16,), t, jnp.int32)])

        def grp_body(j, ng):
            ov = ordered_v[pl.ds(j * 16, 16)]
            key = ov * 16 + iota16
            ks = lax.sort(key)
            sv = jnp.right_shift(ks, 4)
            sl = ks & 15
            tmp_v[...] = sv
            prev = plsc.load_gather(tmp_v, [(iota16 + 15) & 15])
            dup_sorted = jnp.where((sv == prev) & (iota16 >= 1), 1, 0)
            plsc.store_scatter(tmp_v, [sl], dup_sorted)
            dup = tmp_v[...] != 0
            g0 = plsc.load_gather(gtab_v, [ov])
            avail = (g0 == -1) & (~dup)
            avail_i = avail.astype(jnp.int32)
            excl = plsc.cumsum(avail_i) - avail_i
            gid_new = ng + excl
            plsc.store_scatter(gtab_v, [ov], gid_new, mask=avail)
            g1 = plsc.load_gather(gtab_v, [ov])
            r_vec = iota16 + j * 16
            group_v[pl.ds(j * 16, 16)] = jnp.where(r_vec < vc_t, g1, -1)
            return ng + jnp.max(plsc.all_reduce_population_count(avail))

        lax.fori_loop(0, S // 16, grp_body, jnp.int32(0))

        pltpu.sync_copy(sorted_v, sorted_hbm.at[t])
        pltpu.sync_copy(group_v, group_hbm.at[t])


@jax.jit
def kernel(article_sentences, article_sentences_lengths, attention, num_codes):
    B, S_, L_ = article_sentences.shape
    C = attention.shape[1]
    sent = article_sentences.astype(jnp.int32)
    sent_t = jnp.swapaxes(sent, 1, 2)
    lengths = article_sentences_lengths.astype(jnp.int32)
    len_col = lengths.reshape(B, S_, 1)

    grid = (B // BB,)
    row_spec = pl.BlockSpec((BB, C, 1, S_), lambda b: (b, 0, 0, 0))
    att_s, rank_s, cid_s, vc_s = pl.pallas_call(
        _rank_kernel,
        grid=grid,
        in_specs=[
            pl.BlockSpec(memory_space=pltpu.SMEM),
            pl.BlockSpec((BB, C, S_, T), lambda b: (b, 0, 0, 0)),
            pl.BlockSpec((BB, S_, L_), lambda b: (b, 0, 0)),
            pl.BlockSpec((BB, L_, S_), lambda b: (b, 0, 0)),
            pl.BlockSpec((BB, S_, 1), lambda b: (b, 0, 0)),
        ],
        out_specs=(row_spec, row_spec,
                   pl.BlockSpec((BB, 1, S_), lambda b: (b, 0, 0)),
                   pl.BlockSpec((BB, C, 1, 1), lambda b: (b, 0, 0, 0))),
        out_shape=(
            jax.ShapeDtypeStruct((B, C, 1, S_), jnp.float32),
            jax.ShapeDtypeStruct((B, C, 1, S_), jnp.int32),
            jax.ShapeDtypeStruct((B, 1, S_), jnp.int32),
            jax.ShapeDtypeStruct((B, C, 1, 1), jnp.int32),
        ),
    )(num_codes.astype(jnp.int32), attention, sent, sent_t, len_col)

    rank2 = rank_s.reshape(B * C, S_)
    cid2 = cid_s.reshape(B, S_)
    vc2 = vc_s.reshape(B * C)

    sc_fn = pl.kernel(
        _sc_group_kernel,
        out_type=(
            jax.ShapeDtypeStruct((B * C, S_), jnp.int32),
            jax.ShapeDtypeStruct((B * C, S_), jnp.int32),
        ),
        mesh=plsc.VectorSubcoreMesh(core_axis_name="c", subcore_axis_name="s",
                                    num_cores=NC, num_subcores=NS),
        scratch_types=[
            pltpu.VMEM((S_,), jnp.int32),
            pltpu.VMEM((S_,), jnp.int32),
            pltpu.VMEM((S_,), jnp.int32),
            pltpu.VMEM((S_,), jnp.int32),
            pltpu.VMEM((S_,), jnp.int32),
            pltpu.VMEM((S_,), jnp.int32),
            pltpu.VMEM((16,), jnp.int32),
            pltpu.VMEM((B * C,), jnp.int32),
        ],
        compiler_params=pltpu.CompilerParams(needs_layout_passes=False),
    )
    sorted2, group2 = sc_fn(rank2, cid2, vc2)

    return (att_s.reshape(B, C, S_), sorted2.reshape(B, C, S_),
            group2.reshape(B, C, S_))
